# Initial kernel scaffold; baseline (speedup 1.0000x reference)
import jax
import jax.numpy as jnp
from jax.experimental import pallas as pl


def kernel(vertices, onehot, d0, w1, b1, d1, w2, b2, d2, w3, b3, d3, w4, b4, d4, wc1, bc1, wc2, bc2, wc3, bc3):
    raise NotImplementedError("write your pallas kernel here")


# speedup vs baseline: 6.0736x; 6.0736x over previous
import functools

import jax
import jax.numpy as jnp
from jax import lax
from jax.experimental import pallas as pl
from jax.experimental.pallas import tpu as pltpu
from jax.experimental.pallas import tpu_sc as plsc

_NBR = 20
_F32 = jnp.float32


def _knn_body(q_ref, st_ref, o_ref, *, k, w, nearest):
    q = q_ref[0]
    st = st_ref[0]
    inner = jnp.dot(q, st)
    t0 = st[0:1, :]
    t1 = st[1:2, :]
    t2 = st[2:3, :]
    s2 = (t0 * t0 + t1 * t1) + t2 * t2
    q0 = q[:, 0:1]
    q1 = q[:, 1:2]
    q2c = q[:, 2:3]
    q2 = (q0 * q0 + q1 * q1) + q2c * q2c
    if nearest:
        dist = (s2 + q2) - 2.0 * inner
    else:
        dist = (-2.0 * inner + s2) + q2
    rb = q.shape[0]
    iota = lax.broadcasted_iota(jnp.int32, (rb, w), 1)

    cols = []
    d = dist
    for _ in range(k):
        m = jnp.min(d, axis=1, keepdims=True)
        idx = jnp.min(jnp.where(d == m, iota, w), axis=1, keepdims=True)
        cols.append(idx)
        d = jnp.where(iota == idx, jnp.float32(jnp.inf), d)
    out = jnp.concatenate(cols + [cols[-1]] * (32 - k), axis=1)
    o_ref[0] = out


def _knn_ids(qp, stp, k, nearest):
    b, qn, _ = qp.shape
    w = stp.shape[2]
    rb = min(qn, 256)
    return pl.pallas_call(
        functools.partial(_knn_body, k=k, w=w, nearest=nearest),
        grid=(b, qn // rb),
        in_specs=[
            pl.BlockSpec((1, rb, 16), lambda bi, i: (bi, i, 0)),
            pl.BlockSpec((1, 16, w), lambda bi, i: (bi, 0, 0)),
        ],
        out_specs=pl.BlockSpec((1, rb, 32), lambda bi, i: (bi, i, 0)),
        out_shape=jax.ShapeDtypeStruct((b, qn, 32), jnp.int32),
    )(qp, stp)


def _mm_body(x_ref, w_ref, b_ref, o_ref):
    o_ref[...] = jnp.dot(x_ref[...], w_ref[...]) + b_ref[...]


def _matmul(x, w, bvec):
    m, kdim = x.shape
    n = w.shape[1]
    mb = min(m, 256)
    return pl.pallas_call(
        _mm_body,
        grid=(m // mb,),
        in_specs=[
            pl.BlockSpec((mb, kdim), lambda i: (i, 0)),
            pl.BlockSpec((kdim, n), lambda i: (0, 0)),
            pl.BlockSpec((1, n), lambda i: (0, 0)),
        ],
        out_specs=pl.BlockSpec((mb, n), lambda i: (i, 0)),
        out_shape=jax.ShapeDtypeStruct((m, n), _F32),
    )(x, w, bvec.reshape(1, n))


def _theta_n(xyz_n, px, py, pz, sx, sy, sz):
    dx = xyz_n[:, 0:1] - px
    dy = xyz_n[:, 1:2] - py
    dz = xyz_n[:, 2:3] - pz
    n2 = (dx * dx + dy * dy) + dz * dz
    inv = 1.0 / jnp.maximum(jnp.sqrt(n2), 1e-12)
    th = ((dx * inv) * sx + (dy * inv) * sy) + (dz * inv) * sz
    return jnp.maximum(th, 0.0)


def _surf_body(xyz_ref, self_ref, sd_ref, o_ref, *, nn, c):
    px = self_ref[:, 0:1]
    py = self_ref[:, 1:2]
    pz = self_ref[:, 2:3]
    sx = sd_ref[0:1, :]
    sy = sd_ref[1:2, :]
    sz = sd_ref[2:3, :]
    acc = jnp.full((self_ref.shape[0], c), -jnp.inf, _F32)
    for n in range(nn):
        acc = jnp.maximum(acc, _theta_n(xyz_ref[n], px, py, pz, sx, sy, sz))
    o_ref[...] = jnp.maximum(acc, 0.0)


def _conv_body(xyz_ref, self_ref, sd_ref, feat_ref, ctr_ref, o_ref, *, nn, c,
               relu):
    px = self_ref[:, 0:1]
    py = self_ref[:, 1:2]
    pz = self_ref[:, 2:3]
    sx = sd_ref[0:1, :]
    sy = sd_ref[1:2, :]
    sz = sd_ref[2:3, :]
    acc = jnp.full((self_ref.shape[0], c), -jnp.inf, _F32)
    for n in range(nn):
        th = _theta_n(xyz_ref[n], px, py, pz, sx, sy, sz)
        acc = jnp.maximum(acc, th * feat_ref[n])
    out = ctr_ref[...] + acc
    if relu:
        out = jnp.maximum(out, 0.0)
    o_ref[...] = out


def _surf(xyz_g, selfxyz, sd):
    m = selfxyz.shape[0]
    c = sd.shape[1]
    rb = min(m, 256)
    return pl.pallas_call(
        functools.partial(_surf_body, nn=_NBR, c=c),
        grid=(m // rb,),
        in_specs=[
            pl.BlockSpec((_NBR, rb, 16), lambda i: (0, i, 0)),
            pl.BlockSpec((rb, 16), lambda i: (i, 0)),
            pl.BlockSpec((8, c), lambda i: (0, 0)),
        ],
        out_specs=pl.BlockSpec((rb, c), lambda i: (i, 0)),
        out_shape=jax.ShapeDtypeStruct((m, c), _F32),
    )(xyz_g, selfxyz, sd)


def _conv(xyz_g, selfxyz, sd, feat_g, ctr, relu):
    m, c = ctr.shape
    rb = min(m, 256)
    return pl.pallas_call(
        functools.partial(_conv_body, nn=_NBR, c=c, relu=relu),
        grid=(m // rb,),
        in_specs=[
            pl.BlockSpec((_NBR, rb, 16), lambda i: (0, i, 0)),
            pl.BlockSpec((rb, 16), lambda i: (i, 0)),
            pl.BlockSpec((8, c), lambda i: (0, 0)),
            pl.BlockSpec((_NBR, rb, c), lambda i: (0, i, 0)),
            pl.BlockSpec((rb, c), lambda i: (i, 0)),
        ],
        out_specs=pl.BlockSpec((rb, c), lambda i: (i, 0)),
        out_shape=jax.ShapeDtypeStruct((m, c), _F32),
    )(xyz_g, selfxyz, sd, feat_g, ctr)


def _maxpool_body(f_ref, o_ref, *, nn):
    acc = f_ref[0]
    for n in range(1, nn):
        acc = jnp.maximum(acc, f_ref[n])
    o_ref[...] = acc


def _maxpool(feat_g):
    nn, m, c = feat_g.shape
    rb = min(m, 256)
    return pl.pallas_call(
        functools.partial(_maxpool_body, nn=nn),
        grid=(m // rb,),
        in_specs=[pl.BlockSpec((nn, rb, c), lambda i: (0, i, 0))],
        out_specs=pl.BlockSpec((rb, c), lambda i: (i, 0)),
        out_shape=jax.ShapeDtypeStruct((m, c), _F32),
    )(feat_g)


def _rowmax_body(x_ref, o_ref, *, b):
    for i in range(b):
        o_ref[pl.ds(i, 1), :] = jnp.max(x_ref[i], axis=0, keepdims=True)


def _rowmax(x):
    b, r, c = x.shape
    return pl.pallas_call(
        functools.partial(_rowmax_body, b=b),
        out_shape=jax.ShapeDtypeStruct((b, c), _F32),
    )(x)


def _head_body(x_ref, w1_ref, b1_ref, w2_ref, b2_ref, w3_ref, b3_ref, o_ref):
    h = jnp.maximum(jnp.dot(x_ref[...], w1_ref[...]) + b1_ref[...], 0.0)
    h = jnp.maximum(jnp.dot(h, w2_ref[...]) + b2_ref[...], 0.0)
    o_ref[...] = jnp.dot(h, w3_ref[...]) + b3_ref[...]


def _head(x, w1t, b1, w2t, b2, w3t, b3):
    m, kdim = x.shape
    n1 = w1t.shape[1]
    n3 = w3t.shape[1]
    mb = 256
    return pl.pallas_call(
        _head_body,
        grid=(m // mb,),
        in_specs=[
            pl.BlockSpec((mb, kdim), lambda i: (i, 0)),
            pl.BlockSpec((kdim, n1), lambda i: (0, 0)),
            pl.BlockSpec((1, n1), lambda i: (0, 0)),
            pl.BlockSpec((n1, n1), lambda i: (0, 0)),
            pl.BlockSpec((1, n1), lambda i: (0, 0)),
            pl.BlockSpec((n1, n3), lambda i: (0, 0)),
            pl.BlockSpec((1, n3), lambda i: (0, 0)),
        ],
        out_specs=pl.BlockSpec((mb, n3), lambda i: (i, 0)),
        out_shape=jax.ShapeDtypeStruct((m, n3), _F32),
    )(x, w1t, b1.reshape(1, n1), w2t, b2.reshape(1, n1), w3t,
      b3.reshape(1, n3))


_SC_NC = 2
_SC_NS = 16


@functools.lru_cache(maxsize=None)
def _sc_gather_fn(t_rows, d_cols, m_rows):
    nw = _SC_NC * _SC_NS
    rows_pw = m_rows // nw
    assert rows_pw * nw == m_rows and rows_pw % 8 == 0
    ch = 64
    while rows_pw % ch:
        ch //= 2
    chunks = rows_pw // ch
    mesh = plsc.VectorSubcoreMesh(core_axis_name="c", subcore_axis_name="s")

    def body(table_hbm, idx_hbm, out_hbm, idx_v, rows_v, sem):
        wid = lax.axis_index("s") * _SC_NC + lax.axis_index("c")
        base = wid * rows_pw

        def step(g, carry):
            off = base + g * ch
            pltpu.sync_copy(idx_hbm.at[pl.ds(off, ch)], idx_v)
            pltpu.async_copy(table_hbm.at[idx_v], rows_v, sem).wait()
            pltpu.sync_copy(rows_v, out_hbm.at[pl.ds(off, ch)])
            return carry

        lax.fori_loop(0, chunks, step, 0)

    return pl.kernel(
        body,
        out_type=jax.ShapeDtypeStruct((m_rows, d_cols), _F32),
        mesh=mesh,
        scratch_types=[
            pltpu.VMEM((ch,), jnp.int32),
            pltpu.VMEM((ch, d_cols), _F32),
            pltpu.SemaphoreType.DMA,
        ],
        compiler_params=pltpu.CompilerParams(use_tc_tiling_on_sc=False),
    )


def _sc_gather(table, idx):
    fn = _sc_gather_fn(table.shape[0], table.shape[1], idx.shape[0])
    return fn(table, idx)


def _dirnorm_pad(d):
    n = jnp.sqrt(jnp.sum(d * d, axis=0, keepdims=True))
    sd = d / jnp.maximum(n, 1e-12)
    return jnp.pad(sd, ((0, 5), (0, 0)))


def _flat_nbr_idx(ni, rows_per_batch):
    b = ni.shape[0]
    offs = (jnp.arange(b, dtype=jnp.int32) * rows_per_batch)[:, None, None]
    return jnp.transpose(ni + offs, (2, 0, 1)).reshape(-1)


def kernel(vertices, onehot, d0, w1, b1, d1, w2, b2, d2, w3, b3, d3, w4, b4,
           d4, wc1, bc1, wc2, bc2, wc3, bc3):
    b, v, _ = vertices.shape
    vp = jnp.pad(vertices.reshape(b * v, 3), ((0, 0), (0, 13)))
    q1p = vp.reshape(b, v, 16)
    st1 = jnp.transpose(q1p, (0, 2, 1))

    ids1 = _knn_ids(q1p, st1, _NBR + 1, nearest=False)
    ni1 = ids1[:, :, 1:_NBR + 1]
    gi1 = _flat_nbr_idx(ni1, v)
    xyz_g1 = _sc_gather(vp, gi1).reshape(_NBR, b * v, 16)

    fm0 = _surf(xyz_g1, vp, _dirnorm_pad(d0))

    fo1 = _matmul(fm0, w1, b1)
    feat_g1 = _sc_gather(fo1[:, 128:], gi1).reshape(_NBR, b * v, 128)
    fm1 = _conv(xyz_g1, vp, _dirnorm_pad(d1), feat_g1, fo1[:, :128], True)

    v1n = v // 4
    samp1 = jax.random.permutation(jax.random.key(1), v)[:v1n]
    ni4 = ids1[:, samp1, 1:5]
    pg1 = _sc_gather(fm1, _flat_nbr_idx(ni4, v)).reshape(4, b * v1n, 128)
    fp1 = _maxpool(pg1)
    v1p_3d = q1p[:, samp1, :]
    v1p = v1p_3d.reshape(b * v1n, 16)
    st2 = jnp.transpose(v1p_3d, (0, 2, 1))

    ids2 = _knn_ids(v1p_3d, st2, _NBR + 1, nearest=False)
    ni2 = ids2[:, :, 1:_NBR + 1]
    gi2 = _flat_nbr_idx(ni2, v1n)
    xyz_g2 = _sc_gather(v1p, gi2).reshape(_NBR, b * v1n, 16)

    fo2 = _matmul(fp1, w2, b2)
    feat_g2 = _sc_gather(fo2[:, 256:], gi2).reshape(_NBR, b * v1n, 256)
    fm2 = _conv(xyz_g2, v1p, _dirnorm_pad(d2), feat_g2, fo2[:, :256], True)

    fo3 = _matmul(fm2, w3, b3)
    feat_g3 = _sc_gather(fo3[:, 256:], gi2).reshape(_NBR, b * v1n, 256)
    fm3 = _conv(xyz_g2, v1p, _dirnorm_pad(d3), feat_g3, fo3[:, :256], True)

    v2n = v1n // 4
    samp2 = jax.random.permutation(jax.random.key(2), v1n)[:v2n]
    ni4b = ids2[:, samp2, 1:5]
    pg2 = _sc_gather(fm3, _flat_nbr_idx(ni4b, v1n)).reshape(4, b * v2n, 256)
    fp2 = _maxpool(pg2)
    v2p_3d = v1p_3d[:, samp2, :]
    v2p = v2p_3d.reshape(b * v2n, 16)
    st3 = jnp.transpose(v2p_3d, (0, 2, 1))

    ids3 = _knn_ids(v2p_3d, st3, _NBR + 1, nearest=False)
    ni3 = ids3[:, :, 1:_NBR + 1]
    gi3 = _flat_nbr_idx(ni3, v2n)
    xyz_g3 = _sc_gather(v2p, gi3).reshape(_NBR, b * v2n, 16)
    fo4 = _matmul(fp2, w4, b4)
    feat_g4 = _sc_gather(fo4[:, 512:], gi3).reshape(_NBR, b * v2n, 512)
    fm4 = _conv(xyz_g3, v2p, _dirnorm_pad(d4), feat_g4, fo4[:, :512], False)

    f_global = _rowmax(fm4.reshape(b, v2n, 512))

    np1 = _knn_ids(q1p, st2, 1, nearest=True)[:, :, 0]
    np2 = _knn_ids(q1p, st3, 1, nearest=True)[:, :, 0]
    giu1 = _flat_nbr_idx(np1[:, :, None], v1n)
    giu2 = _flat_nbr_idx(np2[:, :, None], v2n)
    fm2u = _sc_gather(fm2, giu1)
    fm3u = _sc_gather(fm3, giu1)
    fm4u = _sc_gather(fm4, giu2)

    fg = jnp.broadcast_to(f_global[:, None, :], (b, v, 512)).reshape(b * v, 512)
    oh = jnp.broadcast_to(onehot[:, None, :], (b, v, 16)).reshape(b * v, 16)
    fuse = jnp.concatenate([fm0, fm1, fm2u, fm3u, fm4u, fg, oh], axis=1)
    w3t = jnp.pad(wc3.T, ((0, 0), (0, 14)))
    b3p = jnp.pad(bc3, (0, 14))
    pred = _head(fuse, wc1.T, bc1, wc2.T, bc2, w3t, b3p)[:, :50]
    return pred.reshape(b, v, 50)


# speedup vs baseline: 6.6499x; 1.0949x over previous
import functools

import jax
import jax.numpy as jnp
from jax import lax
from jax.experimental import pallas as pl
from jax.experimental.pallas import tpu as pltpu
from jax.experimental.pallas import tpu_sc as plsc

_NBR = 20
_F32 = jnp.float32


def _knn_body(q_ref, st_ref, o_ref, *, k, w, nearest):
    q = q_ref[0]
    st = st_ref[0]
    inner = jnp.dot(q, st)
    t0 = st[0:1, :]
    t1 = st[1:2, :]
    t2 = st[2:3, :]
    s2 = (t0 * t0 + t1 * t1) + t2 * t2
    q0 = q[:, 0:1]
    q1 = q[:, 1:2]
    q2c = q[:, 2:3]
    q2 = (q0 * q0 + q1 * q1) + q2c * q2c
    if nearest:
        dist = (s2 + q2) - 2.0 * inner
    else:
        dist = (-2.0 * inner + s2) + q2
    rb = q.shape[0]
    iota = lax.broadcasted_iota(jnp.int32, (rb, w), 1)

    cols = []
    d = dist
    for _ in range(k):
        m = jnp.min(d, axis=1, keepdims=True)
        idx = jnp.min(jnp.where(d == m, iota, w), axis=1, keepdims=True)
        cols.append(idx)
        d = jnp.where(iota == idx, jnp.float32(jnp.inf), d)
    out = jnp.concatenate(cols + [cols[-1]] * (32 - k), axis=1)
    o_ref[0] = out


def _knn_ids(qp, stp, k, nearest):
    b, qn, _ = qp.shape
    w = stp.shape[2]
    rb = min(qn, 256)
    return pl.pallas_call(
        functools.partial(_knn_body, k=k, w=w, nearest=nearest),
        grid=(b, qn // rb),
        in_specs=[
            pl.BlockSpec((1, rb, 16), lambda bi, i: (bi, i, 0)),
            pl.BlockSpec((1, 16, w), lambda bi, i: (bi, 0, 0)),
        ],
        out_specs=pl.BlockSpec((1, rb, 32), lambda bi, i: (bi, i, 0)),
        out_shape=jax.ShapeDtypeStruct((b, qn, 32), jnp.int32),
    )(qp, stp)


def _mm_body(x_ref, w_ref, b_ref, o_ref):
    o_ref[...] = jnp.dot(x_ref[...], w_ref[...]) + b_ref[...]


def _matmul(x, w, bvec):
    m, kdim = x.shape
    n = w.shape[1]
    mb = min(m, 256)
    return pl.pallas_call(
        _mm_body,
        grid=(m // mb,),
        in_specs=[
            pl.BlockSpec((mb, kdim), lambda i: (i, 0)),
            pl.BlockSpec((kdim, n), lambda i: (0, 0)),
            pl.BlockSpec((1, n), lambda i: (0, 0)),
        ],
        out_specs=pl.BlockSpec((mb, n), lambda i: (i, 0)),
        out_shape=jax.ShapeDtypeStruct((m, n), _F32),
    )(x, w, bvec.reshape(1, n))


def _theta_n(xyz_n, px, py, pz, sx, sy, sz):
    dx = xyz_n[:, 0:1] - px
    dy = xyz_n[:, 1:2] - py
    dz = xyz_n[:, 2:3] - pz
    n2 = (dx * dx + dy * dy) + dz * dz
    inv = 1.0 / jnp.maximum(jnp.sqrt(n2), 1e-12)
    th = ((dx * inv) * sx + (dy * inv) * sy) + (dz * inv) * sz
    return jnp.maximum(th, 0.0)


def _surf_body(xyz_ref, self_ref, sd_ref, o_ref, *, nn, c):
    px = self_ref[:, 0:1]
    py = self_ref[:, 1:2]
    pz = self_ref[:, 2:3]
    sx = sd_ref[0:1, :]
    sy = sd_ref[1:2, :]
    sz = sd_ref[2:3, :]
    acc = jnp.full((self_ref.shape[0], c), -jnp.inf, _F32)
    for n in range(nn):
        acc = jnp.maximum(acc, _theta_n(xyz_ref[n], px, py, pz, sx, sy, sz))
    o_ref[...] = jnp.maximum(acc, 0.0)


def _conv_body(xyz_ref, self_ref, sd_ref, feat_ref, ctr_ref, o_ref, *, nn, c,
               relu):
    px = self_ref[:, 0:1]
    py = self_ref[:, 1:2]
    pz = self_ref[:, 2:3]
    sx = sd_ref[0:1, :]
    sy = sd_ref[1:2, :]
    sz = sd_ref[2:3, :]
    acc = jnp.full((self_ref.shape[0], c), -jnp.inf, _F32)
    for n in range(nn):
        th = _theta_n(xyz_ref[n], px, py, pz, sx, sy, sz)
        acc = jnp.maximum(acc, th * feat_ref[n])
    out = ctr_ref[...] + acc
    if relu:
        out = jnp.maximum(out, 0.0)
    o_ref[...] = out


def _surf(xyz_g, selfxyz, sd):
    m = selfxyz.shape[0]
    c = sd.shape[1]
    rb = min(m, 256)
    return pl.pallas_call(
        functools.partial(_surf_body, nn=_NBR, c=c),
        grid=(m // rb,),
        in_specs=[
            pl.BlockSpec((_NBR, rb, 16), lambda i: (0, i, 0)),
            pl.BlockSpec((rb, 16), lambda i: (i, 0)),
            pl.BlockSpec((8, c), lambda i: (0, 0)),
        ],
        out_specs=pl.BlockSpec((rb, c), lambda i: (i, 0)),
        out_shape=jax.ShapeDtypeStruct((m, c), _F32),
    )(xyz_g, selfxyz, sd)


def _conv(xyz_g, selfxyz, sd, feat_g, ctr, relu):
    m, c = ctr.shape
    rb = min(m, 256)
    return pl.pallas_call(
        functools.partial(_conv_body, nn=_NBR, c=c, relu=relu),
        grid=(m // rb,),
        in_specs=[
            pl.BlockSpec((_NBR, rb, 16), lambda i: (0, i, 0)),
            pl.BlockSpec((rb, 16), lambda i: (i, 0)),
            pl.BlockSpec((8, c), lambda i: (0, 0)),
            pl.BlockSpec((_NBR, rb, c), lambda i: (0, i, 0)),
            pl.BlockSpec((rb, c), lambda i: (i, 0)),
        ],
        out_specs=pl.BlockSpec((rb, c), lambda i: (i, 0)),
        out_shape=jax.ShapeDtypeStruct((m, c), _F32),
    )(xyz_g, selfxyz, sd, feat_g, ctr)


def _maxpool_body(f_ref, o_ref, *, nn):
    acc = f_ref[0]
    for n in range(1, nn):
        acc = jnp.maximum(acc, f_ref[n])
    o_ref[...] = acc


def _maxpool(feat_g):
    nn, m, c = feat_g.shape
    rb = min(m, 256)
    return pl.pallas_call(
        functools.partial(_maxpool_body, nn=nn),
        grid=(m // rb,),
        in_specs=[pl.BlockSpec((nn, rb, c), lambda i: (0, i, 0))],
        out_specs=pl.BlockSpec((rb, c), lambda i: (i, 0)),
        out_shape=jax.ShapeDtypeStruct((m, c), _F32),
    )(feat_g)


def _rowmax_body(x_ref, o_ref, *, b):
    for i in range(b):
        o_ref[pl.ds(i, 1), :] = jnp.max(x_ref[i], axis=0, keepdims=True)


def _rowmax(x):
    b, r, c = x.shape
    return pl.pallas_call(
        functools.partial(_rowmax_body, b=b),
        out_shape=jax.ShapeDtypeStruct((b, c), _F32),
    )(x)


def _head_body(x_ref, w1_ref, b1_ref, w2_ref, b2_ref, w3_ref, b3_ref, o_ref):
    h = jnp.maximum(jnp.dot(x_ref[...], w1_ref[...]) + b1_ref[...], 0.0)
    h = jnp.maximum(jnp.dot(h, w2_ref[...]) + b2_ref[...], 0.0)
    o_ref[...] = jnp.dot(h, w3_ref[...]) + b3_ref[...]


def _head(x, w1t, b1, w2t, b2, w3t, b3):
    m, kdim = x.shape
    n1 = w1t.shape[1]
    n3 = w3t.shape[1]
    mb = 256
    return pl.pallas_call(
        _head_body,
        grid=(m // mb,),
        in_specs=[
            pl.BlockSpec((mb, kdim), lambda i: (i, 0)),
            pl.BlockSpec((kdim, n1), lambda i: (0, 0)),
            pl.BlockSpec((1, n1), lambda i: (0, 0)),
            pl.BlockSpec((n1, n1), lambda i: (0, 0)),
            pl.BlockSpec((1, n1), lambda i: (0, 0)),
            pl.BlockSpec((n1, n3), lambda i: (0, 0)),
            pl.BlockSpec((1, n3), lambda i: (0, 0)),
        ],
        out_specs=pl.BlockSpec((mb, n3), lambda i: (i, 0)),
        out_shape=jax.ShapeDtypeStruct((m, n3), _F32),
    )(x, w1t, b1.reshape(1, n1), w2t, b2.reshape(1, n1), w3t,
      b3.reshape(1, n3))


_SC_NC = 2
_SC_NS = 16


def _pick_ch_nbuf(rows_pw, d_cols):
    for nbuf in (4, 3, 2, 1):
        for ch in (128, 64, 32, 16, 8):
            if rows_pw % ch:
                continue
            if (rows_pw // ch) % nbuf:
                continue
            if nbuf * ch * d_cols * 4 > 450 * 1024:
                continue
            return ch, nbuf
    raise ValueError((rows_pw, d_cols))


@functools.lru_cache(maxsize=None)
def _sc_gather_fn(t_rows, d_cols, m_rows):
    nw = _SC_NC * _SC_NS
    rows_pw = m_rows // nw
    assert rows_pw * nw == m_rows and rows_pw % 8 == 0
    ch, nbuf = _pick_ch_nbuf(rows_pw, d_cols)
    chunks = rows_pw // ch
    groups = chunks // nbuf
    mesh = plsc.VectorSubcoreMesh(core_axis_name="c", subcore_axis_name="s")

    def body(table_hbm, idx_hbm, out_hbm, idx_v, rows_v, *sems):
        wid = lax.axis_index("s") * _SC_NC + lax.axis_index("c")
        base = wid * rows_pw

        def fire(g, bi):
            off = base + g * ch
            pltpu.sync_copy(idx_hbm.at[pl.ds(off, ch)], idx_v.at[bi])
            pltpu.async_copy(table_hbm.at[idx_v.at[bi]], rows_v.at[bi],
                             sems[bi])

        def drain_write(g, bi):
            pltpu.make_async_copy(table_hbm.at[idx_v.at[bi]], rows_v.at[bi],
                                  sems[bi]).wait()
            pltpu.sync_copy(rows_v.at[bi], out_hbm.at[pl.ds(base + g * ch, ch)])

        for bi in range(nbuf):
            fire(bi, bi)

        def grp(gp, carry):
            for bi in range(nbuf):
                g = gp * nbuf + bi
                drain_write(g, bi)
                fire(g + nbuf, bi)
            return carry

        lax.fori_loop(0, groups - 1, grp, 0)
        for bi in range(nbuf):
            drain_write((groups - 1) * nbuf + bi, bi)

    return pl.kernel(
        body,
        out_type=jax.ShapeDtypeStruct((m_rows, d_cols), _F32),
        mesh=mesh,
        scratch_types=[
            pltpu.VMEM((nbuf, ch), jnp.int32),
            pltpu.VMEM((nbuf, ch, d_cols), _F32),
        ] + [pltpu.SemaphoreType.DMA] * nbuf,
        compiler_params=pltpu.CompilerParams(use_tc_tiling_on_sc=False),
    )


def _sc_gather(table, idx):
    fn = _sc_gather_fn(table.shape[0], table.shape[1], idx.shape[0])
    return fn(table, idx)


def _dirnorm_pad(d):
    n = jnp.sqrt(jnp.sum(d * d, axis=0, keepdims=True))
    sd = d / jnp.maximum(n, 1e-12)
    return jnp.pad(sd, ((0, 5), (0, 0)))


def _flat_nbr_idx(ni, rows_per_batch):
    b = ni.shape[0]
    offs = (jnp.arange(b, dtype=jnp.int32) * rows_per_batch)[:, None, None]
    return jnp.transpose(ni + offs, (2, 0, 1)).reshape(-1)


def kernel(vertices, onehot, d0, w1, b1, d1, w2, b2, d2, w3, b3, d3, w4, b4,
           d4, wc1, bc1, wc2, bc2, wc3, bc3):
    b, v, _ = vertices.shape
    vp = jnp.pad(vertices.reshape(b * v, 3), ((0, 0), (0, 13)))
    q1p = vp.reshape(b, v, 16)
    st1 = jnp.transpose(q1p, (0, 2, 1))

    ids1 = _knn_ids(q1p, st1, _NBR + 1, nearest=False)
    ni1 = ids1[:, :, 1:_NBR + 1]
    gi1 = _flat_nbr_idx(ni1, v)
    xyz_g1 = _sc_gather(vp, gi1).reshape(_NBR, b * v, 16)

    fm0 = _surf(xyz_g1, vp, _dirnorm_pad(d0))

    fo1 = _matmul(fm0, w1, b1)
    feat_g1 = _sc_gather(fo1[:, 128:], gi1).reshape(_NBR, b * v, 128)
    fm1 = _conv(xyz_g1, vp, _dirnorm_pad(d1), feat_g1, fo1[:, :128], True)

    v1n = v // 4
    samp1 = jax.random.permutation(jax.random.key(1), v)[:v1n]
    ni4 = ids1[:, samp1, 1:5]
    pg1 = _sc_gather(fm1, _flat_nbr_idx(ni4, v)).reshape(4, b * v1n, 128)
    fp1 = _maxpool(pg1)
    v1p_3d = q1p[:, samp1, :]
    v1p = v1p_3d.reshape(b * v1n, 16)
    st2 = jnp.transpose(v1p_3d, (0, 2, 1))

    ids2 = _knn_ids(v1p_3d, st2, _NBR + 1, nearest=False)
    ni2 = ids2[:, :, 1:_NBR + 1]
    gi2 = _flat_nbr_idx(ni2, v1n)
    xyz_g2 = _sc_gather(v1p, gi2).reshape(_NBR, b * v1n, 16)

    fo2 = _matmul(fp1, w2, b2)
    feat_g2 = _sc_gather(fo2[:, 256:], gi2).reshape(_NBR, b * v1n, 256)
    fm2 = _conv(xyz_g2, v1p, _dirnorm_pad(d2), feat_g2, fo2[:, :256], True)

    fo3 = _matmul(fm2, w3, b3)
    feat_g3 = _sc_gather(fo3[:, 256:], gi2).reshape(_NBR, b * v1n, 256)
    fm3 = _conv(xyz_g2, v1p, _dirnorm_pad(d3), feat_g3, fo3[:, :256], True)

    v2n = v1n // 4
    samp2 = jax.random.permutation(jax.random.key(2), v1n)[:v2n]
    ni4b = ids2[:, samp2, 1:5]
    pg2 = _sc_gather(fm3, _flat_nbr_idx(ni4b, v1n)).reshape(4, b * v2n, 256)
    fp2 = _maxpool(pg2)
    v2p_3d = v1p_3d[:, samp2, :]
    v2p = v2p_3d.reshape(b * v2n, 16)
    st3 = jnp.transpose(v2p_3d, (0, 2, 1))

    ids3 = _knn_ids(v2p_3d, st3, _NBR + 1, nearest=False)
    ni3 = ids3[:, :, 1:_NBR + 1]
    gi3 = _flat_nbr_idx(ni3, v2n)
    xyz_g3 = _sc_gather(v2p, gi3).reshape(_NBR, b * v2n, 16)
    fo4 = _matmul(fp2, w4, b4)
    feat_g4 = _sc_gather(fo4[:, 512:], gi3).reshape(_NBR, b * v2n, 512)
    fm4 = _conv(xyz_g3, v2p, _dirnorm_pad(d4), feat_g4, fo4[:, :512], False)

    f_global = _rowmax(fm4.reshape(b, v2n, 512))

    np1 = _knn_ids(q1p, st2, 1, nearest=True)[:, :, 0]
    np2 = _knn_ids(q1p, st3, 1, nearest=True)[:, :, 0]
    giu1 = _flat_nbr_idx(np1[:, :, None], v1n)
    giu2 = _flat_nbr_idx(np2[:, :, None], v2n)
    fm2u = _sc_gather(fm2, giu1)
    fm3u = _sc_gather(fm3, giu1)
    fm4u = _sc_gather(fm4, giu2)

    fg = jnp.broadcast_to(f_global[:, None, :], (b, v, 512)).reshape(b * v, 512)
    oh = jnp.broadcast_to(onehot[:, None, :], (b, v, 16)).reshape(b * v, 16)
    fuse = jnp.concatenate([fm0, fm1, fm2u, fm3u, fm4u, fg, oh], axis=1)
    w3t = jnp.pad(wc3.T, ((0, 0), (0, 14)))
    b3p = jnp.pad(bc3, (0, 14))
    pred = _head(fuse, wc1.T, bc1, wc2.T, bc2, w3t, b3p)[:, :50]
    return pred.reshape(b, v, 50)


# speedup vs baseline: 8.9599x; 1.3474x over previous
import functools

import jax
import jax.numpy as jnp
from jax import lax
from jax.experimental import pallas as pl
from jax.experimental.pallas import tpu as pltpu
from jax.experimental.pallas import tpu_sc as plsc

_NBR = 20
_F32 = jnp.float32


def _knn_body(q_ref, st_ref, o_ref, *, k, w, nearest):
    q = q_ref[0]
    st = st_ref[0]
    inner = jnp.dot(q, st)
    t0 = st[0:1, :]
    t1 = st[1:2, :]
    t2 = st[2:3, :]
    s2 = (t0 * t0 + t1 * t1) + t2 * t2
    q0 = q[:, 0:1]
    q1 = q[:, 1:2]
    q2c = q[:, 2:3]
    q2 = (q0 * q0 + q1 * q1) + q2c * q2c
    if nearest:
        dist = (s2 + q2) - 2.0 * inner
    else:
        dist = (-2.0 * inner + s2) + q2
    rb = q.shape[0]
    iota = lax.broadcasted_iota(jnp.int32, (rb, w), 1)

    cols = []
    d = dist
    for _ in range(k):
        m = jnp.min(d, axis=1, keepdims=True)
        idx = jnp.min(jnp.where(d == m, iota, w), axis=1, keepdims=True)
        cols.append(idx)
        d = jnp.where(iota == idx, jnp.float32(jnp.inf), d)
    out = jnp.concatenate(cols + [cols[-1]] * (32 - k), axis=1)
    o_ref[0] = out


def _knn_ids(qp, stp, k, nearest):
    b, qn, _ = qp.shape
    w = stp.shape[2]
    rb = min(qn, 256)
    return pl.pallas_call(
        functools.partial(_knn_body, k=k, w=w, nearest=nearest),
        grid=(b, qn // rb),
        in_specs=[
            pl.BlockSpec((1, rb, 16), lambda bi, i: (bi, i, 0)),
            pl.BlockSpec((1, 16, w), lambda bi, i: (bi, 0, 0)),
        ],
        out_specs=pl.BlockSpec((1, rb, 32), lambda bi, i: (bi, i, 0)),
        out_shape=jax.ShapeDtypeStruct((b, qn, 32), jnp.int32),
    )(qp, stp)


def _mm_body(x_ref, w_ref, b_ref, o_ref):
    o_ref[...] = jnp.dot(x_ref[...], w_ref[...]) + b_ref[...]


def _matmul(x, w, bvec):
    m, kdim = x.shape
    n = w.shape[1]
    mb = min(m, 256)
    return pl.pallas_call(
        _mm_body,
        grid=(m // mb,),
        in_specs=[
            pl.BlockSpec((mb, kdim), lambda i: (i, 0)),
            pl.BlockSpec((kdim, n), lambda i: (0, 0)),
            pl.BlockSpec((1, n), lambda i: (0, 0)),
        ],
        out_specs=pl.BlockSpec((mb, n), lambda i: (i, 0)),
        out_shape=jax.ShapeDtypeStruct((m, n), _F32),
    )(x, w, bvec.reshape(1, n))


def _theta_parts(xyz_ref, self_ref, sd_ref, nn, c):
    rb = self_ref.shape[0]
    diff = xyz_ref[...] - self_ref[...][None, :, :]
    sq = diff * diff
    n2 = (sq[:, :, 0:1] + sq[:, :, 1:2]) + sq[:, :, 2:3]
    n2f = n2.reshape(nn * rb // 128, 128)
    invf = 1.0 / jnp.maximum(jnp.sqrt(n2f), 1e-12)
    inv = invf.reshape(nn, rb, 1)
    t = jnp.dot(diff.reshape(nn * rb, 16), sd_ref[...])
    th = jnp.maximum(t, 0.0).reshape(nn, rb, c)
    return th, inv


def _surf_body(xyz_ref, self_ref, sd_ref, o_ref, *, nn, c):
    th, inv = _theta_parts(xyz_ref, self_ref, sd_ref, nn, c)
    acc = th[0] * inv[0]
    for n in range(1, nn):
        acc = jnp.maximum(acc, th[n] * inv[n])
    o_ref[...] = jnp.maximum(acc, 0.0)


def _conv_body(xyz_ref, self_ref, sd_ref, feat_ref, ctr_ref, o_ref, *, nn, c,
               relu):
    th, inv = _theta_parts(xyz_ref, self_ref, sd_ref, nn, c)
    acc = (th[0] * inv[0]) * feat_ref[0]
    for n in range(1, nn):
        acc = jnp.maximum(acc, (th[n] * inv[n]) * feat_ref[n])
    out = ctr_ref[...] + acc
    if relu:
        out = jnp.maximum(out, 0.0)
    o_ref[...] = out


def _surf(xyz_g, selfxyz, sd):
    m = selfxyz.shape[0]
    c = sd.shape[1]
    rb = min(m, 256)
    return pl.pallas_call(
        functools.partial(_surf_body, nn=_NBR, c=c),
        grid=(m // rb,),
        in_specs=[
            pl.BlockSpec((_NBR, rb, 16), lambda i: (0, i, 0)),
            pl.BlockSpec((rb, 16), lambda i: (i, 0)),
            pl.BlockSpec((16, c), lambda i: (0, 0)),
        ],
        out_specs=pl.BlockSpec((rb, c), lambda i: (i, 0)),
        out_shape=jax.ShapeDtypeStruct((m, c), _F32),
    )(xyz_g, selfxyz, sd)


def _conv(xyz_g, selfxyz, sd, feat_g, ctr, relu):
    m, c = ctr.shape
    rb = min(m, 256)
    return pl.pallas_call(
        functools.partial(_conv_body, nn=_NBR, c=c, relu=relu),
        grid=(m // rb,),
        in_specs=[
            pl.BlockSpec((_NBR, rb, 16), lambda i: (0, i, 0)),
            pl.BlockSpec((rb, 16), lambda i: (i, 0)),
            pl.BlockSpec((16, c), lambda i: (0, 0)),
            pl.BlockSpec((_NBR, rb, c), lambda i: (0, i, 0)),
            pl.BlockSpec((rb, c), lambda i: (i, 0)),
        ],
        out_specs=pl.BlockSpec((rb, c), lambda i: (i, 0)),
        out_shape=jax.ShapeDtypeStruct((m, c), _F32),
    )(xyz_g, selfxyz, sd, feat_g, ctr)


def _maxpool_body(f_ref, o_ref, *, nn):
    acc = f_ref[0]
    for n in range(1, nn):
        acc = jnp.maximum(acc, f_ref[n])
    o_ref[...] = acc


def _maxpool(feat_g):
    nn, m, c = feat_g.shape
    rb = min(m, 256)
    return pl.pallas_call(
        functools.partial(_maxpool_body, nn=nn),
        grid=(m // rb,),
        in_specs=[pl.BlockSpec((nn, rb, c), lambda i: (0, i, 0))],
        out_specs=pl.BlockSpec((rb, c), lambda i: (i, 0)),
        out_shape=jax.ShapeDtypeStruct((m, c), _F32),
    )(feat_g)


def _rowmax_body(x_ref, o_ref, *, b):
    for i in range(b):
        o_ref[pl.ds(i, 1), :] = jnp.max(x_ref[i], axis=0, keepdims=True)


def _rowmax(x):
    b, r, c = x.shape
    return pl.pallas_call(
        functools.partial(_rowmax_body, b=b),
        out_shape=jax.ShapeDtypeStruct((b, c), _F32),
    )(x)


def _head_body(x_ref, w1_ref, b1_ref, w2_ref, b2_ref, w3_ref, b3_ref, o_ref):
    h = jnp.maximum(jnp.dot(x_ref[...], w1_ref[...]) + b1_ref[...], 0.0)
    h = jnp.maximum(jnp.dot(h, w2_ref[...]) + b2_ref[...], 0.0)
    o_ref[...] = jnp.dot(h, w3_ref[...]) + b3_ref[...]


def _head(x, w1t, b1, w2t, b2, w3t, b3):
    m, kdim = x.shape
    n1 = w1t.shape[1]
    n3 = w3t.shape[1]
    mb = 256
    return pl.pallas_call(
        _head_body,
        grid=(m // mb,),
        in_specs=[
            pl.BlockSpec((mb, kdim), lambda i: (i, 0)),
            pl.BlockSpec((kdim, n1), lambda i: (0, 0)),
            pl.BlockSpec((1, n1), lambda i: (0, 0)),
            pl.BlockSpec((n1, n1), lambda i: (0, 0)),
            pl.BlockSpec((1, n1), lambda i: (0, 0)),
            pl.BlockSpec((n1, n3), lambda i: (0, 0)),
            pl.BlockSpec((1, n3), lambda i: (0, 0)),
        ],
        out_specs=pl.BlockSpec((mb, n3), lambda i: (i, 0)),
        out_shape=jax.ShapeDtypeStruct((m, n3), _F32),
    )(x, w1t, b1.reshape(1, n1), w2t, b2.reshape(1, n1), w3t,
      b3.reshape(1, n3))


_SC_NC = 2
_SC_NS = 16


def _pick_ch_nbuf(rows_pw, d_cols):
    for nbuf in (4, 3, 2, 1):
        for ch in (128, 64, 32, 16, 8):
            if rows_pw % ch:
                continue
            if (rows_pw // ch) % nbuf:
                continue
            if nbuf * ch * d_cols * 4 > 450 * 1024:
                continue
            return ch, nbuf
    raise ValueError((rows_pw, d_cols))


@functools.lru_cache(maxsize=None)
def _sc_gather_fn(t_rows, d_cols, m_rows):
    nw = _SC_NC * _SC_NS
    rows_pw = m_rows // nw
    assert rows_pw * nw == m_rows and rows_pw % 8 == 0
    ch, nbuf = _pick_ch_nbuf(rows_pw, d_cols)
    chunks = rows_pw // ch
    groups = chunks // nbuf
    mesh = plsc.VectorSubcoreMesh(core_axis_name="c", subcore_axis_name="s")

    def body(table_hbm, idx_hbm, out_hbm, idx_v, rows_v, *sems):
        wid = lax.axis_index("s") * _SC_NC + lax.axis_index("c")
        base = wid * rows_pw

        def fire(g, bi):
            off = base + g * ch
            pltpu.sync_copy(idx_hbm.at[pl.ds(off, ch)], idx_v.at[bi])
            pltpu.async_copy(table_hbm.at[idx_v.at[bi]], rows_v.at[bi],
                             sems[bi])

        def drain_write(g, bi):
            pltpu.make_async_copy(table_hbm.at[idx_v.at[bi]], rows_v.at[bi],
                                  sems[bi]).wait()
            pltpu.sync_copy(rows_v.at[bi], out_hbm.at[pl.ds(base + g * ch, ch)])

        for bi in range(nbuf):
            fire(bi, bi)

        def grp(gp, carry):
            for bi in range(nbuf):
                g = gp * nbuf + bi
                drain_write(g, bi)
                fire(g + nbuf, bi)
            return carry

        lax.fori_loop(0, groups - 1, grp, 0)
        for bi in range(nbuf):
            drain_write((groups - 1) * nbuf + bi, bi)

    return pl.kernel(
        body,
        out_type=jax.ShapeDtypeStruct((m_rows, d_cols), _F32),
        mesh=mesh,
        scratch_types=[
            pltpu.VMEM((nbuf, ch), jnp.int32),
            pltpu.VMEM((nbuf, ch, d_cols), _F32),
        ] + [pltpu.SemaphoreType.DMA] * nbuf,
        compiler_params=pltpu.CompilerParams(use_tc_tiling_on_sc=False),
    )


def _sc_gather(table, idx):
    fn = _sc_gather_fn(table.shape[0], table.shape[1], idx.shape[0])
    return fn(table, idx)


def _dirnorm_pad(d):
    n = jnp.sqrt(jnp.sum(d * d, axis=0, keepdims=True))
    sd = d / jnp.maximum(n, 1e-12)
    return jnp.pad(sd, ((0, 13), (0, 0)))


def _flat_nbr_idx(ni, rows_per_batch):
    b = ni.shape[0]
    offs = (jnp.arange(b, dtype=jnp.int32) * rows_per_batch)[:, None, None]
    return jnp.transpose(ni + offs, (2, 0, 1)).reshape(-1)


def kernel(vertices, onehot, d0, w1, b1, d1, w2, b2, d2, w3, b3, d3, w4, b4,
           d4, wc1, bc1, wc2, bc2, wc3, bc3):
    b, v, _ = vertices.shape
    vp = jnp.pad(vertices.reshape(b * v, 3), ((0, 0), (0, 13)))
    q1p = vp.reshape(b, v, 16)
    st1 = jnp.transpose(q1p, (0, 2, 1))

    ids1 = _knn_ids(q1p, st1, _NBR + 1, nearest=False)
    ni1 = ids1[:, :, 1:_NBR + 1]
    gi1 = _flat_nbr_idx(ni1, v)
    xyz_g1 = _sc_gather(vp, gi1).reshape(_NBR, b * v, 16)

    fm0 = _surf(xyz_g1, vp, _dirnorm_pad(d0))

    fo1 = _matmul(fm0, w1, b1)
    feat_g1 = _sc_gather(fo1[:, 128:], gi1).reshape(_NBR, b * v, 128)
    fm1 = _conv(xyz_g1, vp, _dirnorm_pad(d1), feat_g1, fo1[:, :128], True)

    v1n = v // 4
    samp1 = jax.random.permutation(jax.random.key(1), v)[:v1n]
    ni4 = ids1[:, samp1, 1:5]
    pg1 = _sc_gather(fm1, _flat_nbr_idx(ni4, v)).reshape(4, b * v1n, 128)
    fp1 = _maxpool(pg1)
    v1p_3d = q1p[:, samp1, :]
    v1p = v1p_3d.reshape(b * v1n, 16)
    st2 = jnp.transpose(v1p_3d, (0, 2, 1))

    ids2 = _knn_ids(v1p_3d, st2, _NBR + 1, nearest=False)
    ni2 = ids2[:, :, 1:_NBR + 1]
    gi2 = _flat_nbr_idx(ni2, v1n)
    xyz_g2 = _sc_gather(v1p, gi2).reshape(_NBR, b * v1n, 16)

    fo2 = _matmul(fp1, w2, b2)
    feat_g2 = _sc_gather(fo2[:, 256:], gi2).reshape(_NBR, b * v1n, 256)
    fm2 = _conv(xyz_g2, v1p, _dirnorm_pad(d2), feat_g2, fo2[:, :256], True)

    fo3 = _matmul(fm2, w3, b3)
    feat_g3 = _sc_gather(fo3[:, 256:], gi2).reshape(_NBR, b * v1n, 256)
    fm3 = _conv(xyz_g2, v1p, _dirnorm_pad(d3), feat_g3, fo3[:, :256], True)

    v2n = v1n // 4
    samp2 = jax.random.permutation(jax.random.key(2), v1n)[:v2n]
    ni4b = ids2[:, samp2, 1:5]
    pg2 = _sc_gather(fm3, _flat_nbr_idx(ni4b, v1n)).reshape(4, b * v2n, 256)
    fp2 = _maxpool(pg2)
    v2p_3d = v1p_3d[:, samp2, :]
    v2p = v2p_3d.reshape(b * v2n, 16)
    st3 = jnp.transpose(v2p_3d, (0, 2, 1))

    ids3 = _knn_ids(v2p_3d, st3, _NBR + 1, nearest=False)
    ni3 = ids3[:, :, 1:_NBR + 1]
    gi3 = _flat_nbr_idx(ni3, v2n)
    xyz_g3 = _sc_gather(v2p, gi3).reshape(_NBR, b * v2n, 16)
    fo4 = _matmul(fp2, w4, b4)
    feat_g4 = _sc_gather(fo4[:, 512:], gi3).reshape(_NBR, b * v2n, 512)
    fm4 = _conv(xyz_g3, v2p, _dirnorm_pad(d4), feat_g4, fo4[:, :512], False)

    f_global = _rowmax(fm4.reshape(b, v2n, 512))

    np1 = _knn_ids(q1p, st2, 1, nearest=True)[:, :, 0]
    np2 = _knn_ids(q1p, st3, 1, nearest=True)[:, :, 0]
    giu1 = _flat_nbr_idx(np1[:, :, None], v1n)
    giu2 = _flat_nbr_idx(np2[:, :, None], v2n)
    fm2u = _sc_gather(fm2, giu1)
    fm3u = _sc_gather(fm3, giu1)
    fm4u = _sc_gather(fm4, giu2)

    fg = jnp.broadcast_to(f_global[:, None, :], (b, v, 512)).reshape(b * v, 512)
    oh = jnp.broadcast_to(onehot[:, None, :], (b, v, 16)).reshape(b * v, 16)
    fuse = jnp.concatenate([fm0, fm1, fm2u, fm3u, fm4u, fg, oh], axis=1)
    w3t = jnp.pad(wc3.T, ((0, 0), (0, 14)))
    b3p = jnp.pad(bc3, (0, 14))
    pred = _head(fuse, wc1.T, bc1, wc2.T, bc2, w3t, b3p)[:, :50]
    return pred.reshape(b, v, 50)
